# baseline (speedup 1.0000x reference)
import functools

import jax
import jax.numpy as jnp
from jax.experimental import pallas as pl
from jax.experimental.pallas import tpu as pltpu
from jax.experimental.pallas import tpu_sc as plsc

BETA = 0.25
_BM = 128
_CHUNK = 128


def _dist_argmin_body(z_ref, cb_ref, zn_ref, cn_ref, idx_ref, dmin_ref):
    t = pl.program_id(0)
    hw = z_ref.shape[2]
    n = cb_ref.shape[0]
    tiles_per_batch = hw // _BM
    b = t // tiles_per_batch
    chunk = t % tiles_per_batch

    zs = z_ref[b, :, pl.ds(chunk * _BM, _BM)]
    zt = jnp.transpose(zs)
    dot = jax.lax.dot_general(
        zt, cb_ref[...],
        dimension_numbers=(((1,), (1,)), ((), ())),
        preferred_element_type=jnp.float32,
    )
    zn = jnp.transpose(zn_ref[pl.ds(t, 1), :])
    s = (zn + cn_ref[...]) - 2.0 * dot

    rm = jax.lax.slice(s, (0, 0), (_BM, _CHUNK))
    rc = jnp.zeros((_BM, _CHUNK), jnp.int32)
    for k in range(1, n // _CHUNK):
        v = jax.lax.slice(s, (0, k * _CHUNK), (_BM, (k + 1) * _CHUNK))
        mask = v < rm
        rm = jnp.where(mask, v, rm)
        rc = jnp.where(mask, jnp.int32(k), rc)

    fmin = jnp.min(rm, axis=1, keepdims=True)
    lane = jax.lax.broadcasted_iota(jnp.int32, (_BM, _CHUNK), 1)
    gidx = rc * _CHUNK + lane
    big = jnp.int32(2**31 - 1)
    fidx = jnp.min(jnp.where(rm == fmin, gidx, big), axis=1, keepdims=True)

    idx_ref[...] = fidx
    dmin_ref[...] = fmin


def _dist_argmin(z3, codebook, zn64, c_norm, interpret=False):
    b, c, hw = z3.shape
    m = b * hw
    n = codebook.shape[0]
    grid = (m // _BM,)
    return pl.pallas_call(
        _dist_argmin_body,
        grid=grid,
        in_specs=[
            pl.BlockSpec(z3.shape, lambda t: (0, 0, 0)),
            pl.BlockSpec(codebook.shape, lambda t: (0, 0)),
            pl.BlockSpec(zn64.shape, lambda t: (0, 0)),
            pl.BlockSpec(c_norm.shape, lambda t: (0, 0)),
        ],
        out_specs=[
            pl.BlockSpec((_BM, 1), lambda t: (t, 0)),
            pl.BlockSpec((_BM, 1), lambda t: (t, 0)),
        ],
        out_shape=[
            jax.ShapeDtypeStruct((m, 1), jnp.int32),
            jax.ShapeDtypeStruct((m, 1), jnp.float32),
        ],
        interpret=interpret,
    )(z3, codebook, zn64, c_norm)


def _st_transpose_body(z_ref, zq_ref, out_ref):
    zb = z_ref[0]
    zqt = jnp.transpose(zq_ref[...])
    out_ref[0] = zb + (zqt - zb)


def _st_transpose(z3, z_q_flat, interpret=False):
    b, c, hw = z3.shape
    return pl.pallas_call(
        _st_transpose_body,
        grid=(b,),
        in_specs=[
            pl.BlockSpec((1, c, hw), lambda t: (t, 0, 0)),
            pl.BlockSpec((hw, c), lambda t: (t, 0)),
        ],
        out_specs=pl.BlockSpec((1, c, hw), lambda t: (t, 0, 0)),
        out_shape=jax.ShapeDtypeStruct((b, c, hw), jnp.float32),
        interpret=interpret,
    )(z3, z_q_flat)


_GATHER_W = 128


def _sc_gather(codebook, idx_flat):
    num_idx = idx_flat.shape[0]
    e_dim = codebook.shape[1]
    idx2 = idx_flat.reshape(1, num_idx)
    mesh = plsc.VectorSubcoreMesh(core_axis_name="core",
                                  subcore_axis_name="subcore")

    @functools.partial(
        pl.kernel,
        out_type=jax.ShapeDtypeStruct((num_idx, e_dim), codebook.dtype),
        mesh=mesh,
    )
    def gather_kernel(cb_hbm, i_hbm, o_hbm):
        def body(i_vmem, o_vmem):
            pltpu.sync_copy(cb_hbm.at[i_vmem.at[0]], o_vmem)

        pltpu.emit_pipeline(
            body,
            grid=(num_idx // _GATHER_W,),
            in_specs=[pl.BlockSpec((1, _GATHER_W), index_map=lambda i: (0, i))],
            out_specs=[pl.BlockSpec((_GATHER_W, e_dim),
                                    index_map=lambda i: (i, 0))],
            core_axis_name=("core", "subcore"),
            dimension_semantics=(pltpu.PARALLEL,),
        )(i_hbm, o_hbm)

    return gather_kernel(codebook, idx2)


def kernel(z, codebook):
    b, c, h, w = z.shape
    n_e = codebook.shape[0]
    z3 = z.reshape(b, c, h * w)
    zt = jnp.transpose(z, (0, 2, 3, 1))
    z_flat = zt.reshape(-1, c)
    z_norm = jnp.sum(z_flat ** 2, axis=1, keepdims=True)
    c_norm = jnp.sum(codebook ** 2, axis=1)[None, :]
    zn64 = z_norm.reshape(-1, _BM)

    idx2, dmin = _dist_argmin(z3, codebook, zn64, c_norm)
    idx_flat = idx2.reshape(-1)

    z_q_flat = _sc_gather(codebook, idx_flat)
    z_q = _st_transpose(z3, z_q_flat).reshape(z.shape)

    mean_sq = jnp.sum(dmin) / (b * h * w * c)
    loss = mean_sq + BETA * mean_sq
    idx = idx_flat.reshape(b, h, w)
    return z_q, loss, idx


# speedup vs baseline: 1.0468x; 1.0468x over previous
import functools

import jax
import jax.numpy as jnp
from jax.experimental import pallas as pl
from jax.experimental.pallas import tpu as pltpu
from jax.experimental.pallas import tpu_sc as plsc

BETA = 0.25
_BM = 128
_CHUNK = 128


def _dist_argmin_body(z_ref, cb_ref, zn_ref, cn_ref, idx_ref, dmin_ref):
    t = pl.program_id(0)
    hw = z_ref.shape[2]
    n = cb_ref.shape[0]
    tiles_per_batch = hw // _BM
    b = t // tiles_per_batch
    chunk = t % tiles_per_batch

    zs = z_ref[b, :, pl.ds(chunk * _BM, _BM)]
    dot = jax.lax.dot_general(
        zs, cb_ref[...],
        dimension_numbers=(((0,), (1,)), ((), ())),
        preferred_element_type=jnp.float32,
    )
    zn = jnp.transpose(zn_ref[pl.ds(t, 1), :])

    def s_chunk(k):
        dk = jax.lax.slice(dot, (0, k * _CHUNK), (_BM, (k + 1) * _CHUNK))
        cnk = jax.lax.slice(cn_ref[...], (0, k * _CHUNK), (1, (k + 1) * _CHUNK))
        return (zn + cnk) + dk

    rm = s_chunk(0)
    rc = jnp.zeros((_BM, _CHUNK), jnp.int32)
    for k in range(1, n // _CHUNK):
        v = s_chunk(k)
        mask = v < rm
        rm = jnp.where(mask, v, rm)
        rc = jnp.where(mask, jnp.int32(k), rc)

    fmin = jnp.min(rm, axis=1, keepdims=True)
    lane = jax.lax.broadcasted_iota(jnp.int32, (_BM, _CHUNK), 1)
    gidx = rc * _CHUNK + lane
    big = jnp.int32(2**31 - 1)
    fidx = jnp.min(jnp.where(rm == fmin, gidx, big), axis=1, keepdims=True)

    idx_ref[...] = fidx
    dmin_ref[...] = fmin


def _dist_argmin(z3, codebook, zn64, c_norm, interpret=False):
    b, c, hw = z3.shape
    m = b * hw
    n = codebook.shape[0]
    grid = (m // _BM,)
    return pl.pallas_call(
        _dist_argmin_body,
        grid=grid,
        in_specs=[
            pl.BlockSpec(z3.shape, lambda t: (0, 0, 0)),
            pl.BlockSpec(codebook.shape, lambda t: (0, 0)),
            pl.BlockSpec(zn64.shape, lambda t: (0, 0)),
            pl.BlockSpec(c_norm.shape, lambda t: (0, 0)),
        ],
        out_specs=[
            pl.BlockSpec((_BM, 1), lambda t: (t, 0)),
            pl.BlockSpec((_BM, 1), lambda t: (t, 0)),
        ],
        out_shape=[
            jax.ShapeDtypeStruct((m, 1), jnp.int32),
            jax.ShapeDtypeStruct((m, 1), jnp.float32),
        ],
        interpret=interpret,
    )(z3, codebook, zn64, c_norm)


def _st_transpose_body(z_ref, zq_ref, out_ref):
    zb = z_ref[0]
    zqt = jnp.transpose(zq_ref[...])
    out_ref[0] = zb + (zqt - zb)


def _st_transpose(z3, z_q_flat, interpret=False):
    b, c, hw = z3.shape
    return pl.pallas_call(
        _st_transpose_body,
        grid=(b,),
        in_specs=[
            pl.BlockSpec((1, c, hw), lambda t: (t, 0, 0)),
            pl.BlockSpec((hw, c), lambda t: (t, 0)),
        ],
        out_specs=pl.BlockSpec((1, c, hw), lambda t: (t, 0, 0)),
        out_shape=jax.ShapeDtypeStruct((b, c, hw), jnp.float32),
        interpret=interpret,
    )(z3, z_q_flat)


_GATHER_W = 128


def _sc_gather(codebook, idx_flat):
    num_idx = idx_flat.shape[0]
    e_dim = codebook.shape[1]
    idx2 = idx_flat.reshape(1, num_idx)
    mesh = plsc.VectorSubcoreMesh(core_axis_name="core",
                                  subcore_axis_name="subcore")

    @functools.partial(
        pl.kernel,
        out_type=jax.ShapeDtypeStruct((num_idx, e_dim), codebook.dtype),
        mesh=mesh,
    )
    def gather_kernel(cb_hbm, i_hbm, o_hbm):
        def body(i_vmem, o_vmem):
            pltpu.sync_copy(cb_hbm.at[i_vmem.at[0]], o_vmem)

        pltpu.emit_pipeline(
            body,
            grid=(num_idx // _GATHER_W,),
            in_specs=[pl.BlockSpec((1, _GATHER_W), index_map=lambda i: (0, i))],
            out_specs=[pl.BlockSpec((_GATHER_W, e_dim),
                                    index_map=lambda i: (i, 0))],
            core_axis_name=("core", "subcore"),
            dimension_semantics=(pltpu.PARALLEL,),
        )(i_hbm, o_hbm)

    return gather_kernel(codebook, idx2)


def kernel(z, codebook):
    b, c, h, w = z.shape
    n_e = codebook.shape[0]
    z3 = z.reshape(b, c, h * w)
    zt = jnp.transpose(z, (0, 2, 3, 1))
    z_flat = zt.reshape(-1, c)
    z_norm = jnp.sum(z_flat ** 2, axis=1, keepdims=True)
    c_norm = jnp.sum(codebook ** 2, axis=1)[None, :]
    zn64 = z_norm.reshape(-1, _BM)

    z3_bf = z3.astype(jnp.bfloat16)
    cbs_bf = (-2.0 * codebook).astype(jnp.bfloat16)

    idx2, dmin = _dist_argmin(z3_bf, cbs_bf, zn64, c_norm)
    idx_flat = idx2.reshape(-1)

    z_q_flat = _sc_gather(codebook, idx_flat)
    z_q = _st_transpose(z3, z_q_flat).reshape(z.shape)

    mean_sq = jnp.sum(dmin) / (b * h * w * c)
    loss = mean_sq + BETA * mean_sq
    idx = idx_flat.reshape(b, h, w)
    return z_q, loss, idx


# speedup vs baseline: 1.1436x; 1.0925x over previous
import functools

import jax
import jax.numpy as jnp
from jax.experimental import pallas as pl
from jax.experimental.pallas import tpu as pltpu
from jax.experimental.pallas import tpu_sc as plsc

BETA = 0.25
_BM = 128
_CHUNK = 128


def _dist_argmin_body(z_ref, cbt_ref, zn_ref, cn_ref, idx_ref, dmin_ref):
    t = pl.program_id(0)
    hw = z_ref.shape[2]
    n = cbt_ref.shape[1]
    tiles_per_batch = hw // _BM
    b = t // tiles_per_batch
    chunk = t % tiles_per_batch

    zs = z_ref[b, :, pl.ds(chunk * _BM, _BM)]
    lhs = jnp.transpose(zs).astype(jnp.bfloat16)
    dot = jax.lax.dot_general(
        lhs, cbt_ref[...],
        dimension_numbers=(((1,), (0,)), ((), ())),
        preferred_element_type=jnp.float32,
    )
    zn = jnp.transpose(zn_ref[pl.ds(t, 1), :])

    def s_chunk(k):
        dk = jax.lax.slice(dot, (0, k * _CHUNK), (_BM, (k + 1) * _CHUNK))
        cnk = jax.lax.slice(cn_ref[...], (0, k * _CHUNK), (1, (k + 1) * _CHUNK))
        return (zn + cnk) + dk

    rm = s_chunk(0)
    rc = jnp.zeros((_BM, _CHUNK), jnp.int32)
    for k in range(1, n // _CHUNK):
        v = s_chunk(k)
        mask = v < rm
        rm = jnp.where(mask, v, rm)
        rc = jnp.where(mask, jnp.int32(k), rc)

    fmin = jnp.min(rm, axis=1, keepdims=True)
    lane = jax.lax.broadcasted_iota(jnp.int32, (_BM, _CHUNK), 1)
    gidx = rc * _CHUNK + lane
    big = jnp.int32(2**31 - 1)
    fidx = jnp.min(jnp.where(rm == fmin, gidx, big), axis=1, keepdims=True)

    idx_ref[0, 0, :] = jnp.transpose(fidx).reshape(_BM)
    dmin_ref[0, 0, :] = jnp.transpose(fmin).reshape(_BM)


def _dist_argmin(z3, cbt, zn64, c_norm, interpret=False):
    b, c, hw = z3.shape
    m = b * hw
    n = cbt.shape[1]
    grid = (m // _BM,)
    return pl.pallas_call(
        _dist_argmin_body,
        grid=grid,
        in_specs=[
            pl.BlockSpec(z3.shape, lambda t: (0, 0, 0)),
            pl.BlockSpec(cbt.shape, lambda t: (0, 0)),
            pl.BlockSpec(zn64.shape, lambda t: (0, 0)),
            pl.BlockSpec(c_norm.shape, lambda t: (0, 0)),
        ],
        out_specs=[
            pl.BlockSpec((1, 1, _BM), lambda t: (t, 0, 0)),
            pl.BlockSpec((1, 1, _BM), lambda t: (t, 0, 0)),
        ],
        out_shape=[
            jax.ShapeDtypeStruct((m // _BM, 1, _BM), jnp.int32),
            jax.ShapeDtypeStruct((m // _BM, 1, _BM), jnp.float32),
        ],
        interpret=interpret,
    )(z3, cbt, zn64, c_norm)


_EP_W = 256


def _st_transpose_body(z_ref, zq_ref, out_ref):
    zb = z_ref[0]
    zqt = jnp.transpose(zq_ref[...])
    out_ref[0] = zb + (zqt - zb)


def _st_transpose(z3, z_q_flat, interpret=False):
    b, c, hw = z3.shape
    j_steps = hw // _EP_W
    return pl.pallas_call(
        _st_transpose_body,
        grid=(b, j_steps),
        in_specs=[
            pl.BlockSpec((1, c, _EP_W), lambda t, j: (t, 0, j)),
            pl.BlockSpec((_EP_W, c), lambda t, j: (t * j_steps + j, 0)),
        ],
        out_specs=pl.BlockSpec((1, c, _EP_W), lambda t, j: (t, 0, j)),
        out_shape=jax.ShapeDtypeStruct((b, c, hw), jnp.float32),
        interpret=interpret,
    )(z3, z_q_flat)


_GATHER_W = 128


def _sc_gather(codebook, idx_flat):
    num_idx = idx_flat.shape[0]
    e_dim = codebook.shape[1]
    idx2 = idx_flat.reshape(1, num_idx)
    mesh = plsc.VectorSubcoreMesh(core_axis_name="core",
                                  subcore_axis_name="subcore")

    @functools.partial(
        pl.kernel,
        out_type=jax.ShapeDtypeStruct((num_idx, e_dim), codebook.dtype),
        mesh=mesh,
    )
    def gather_kernel(cb_hbm, i_hbm, o_hbm):
        def body(i_vmem, o_vmem):
            pltpu.sync_copy(cb_hbm.at[i_vmem.at[0]], o_vmem)

        pltpu.emit_pipeline(
            body,
            grid=(num_idx // _GATHER_W,),
            in_specs=[pl.BlockSpec((1, _GATHER_W), index_map=lambda i: (0, i))],
            out_specs=[pl.BlockSpec((_GATHER_W, e_dim),
                                    index_map=lambda i: (i, 0))],
            core_axis_name=("core", "subcore"),
            dimension_semantics=(pltpu.PARALLEL,),
        )(i_hbm, o_hbm)

    return gather_kernel(codebook, idx2)


def kernel(z, codebook):
    b, c, h, w = z.shape
    n_e = codebook.shape[0]
    z3 = z.reshape(b, c, h * w)
    zt = jnp.transpose(z, (0, 2, 3, 1))
    z_flat = zt.reshape(-1, c)
    z_norm = jnp.sum(z_flat ** 2, axis=1, keepdims=True)
    c_norm = jnp.sum(codebook ** 2, axis=1)[None, :]
    zn64 = z_norm.reshape(-1, _BM)

    cbt_bf = jnp.transpose((-2.0 * codebook).astype(jnp.bfloat16))

    idx3, dmin3 = _dist_argmin(z3, cbt_bf, zn64, c_norm)
    idx_flat = idx3.reshape(-1)

    z_q_flat = _sc_gather(codebook, idx_flat)
    z_q = _st_transpose(z3, z_q_flat).reshape(z.shape)

    mean_sq = jnp.sum(dmin3) / (b * h * w * c)
    loss = mean_sq + BETA * mean_sq
    idx = idx_flat.reshape(b, h, w)
    return z_q, loss, idx


# speedup vs baseline: 1.1926x; 1.0428x over previous
import functools

import jax
import jax.numpy as jnp
from jax.experimental import pallas as pl
from jax.experimental.pallas import tpu as pltpu
from jax.experimental.pallas import tpu_sc as plsc

BETA = 0.25
_BM = 128
_CHUNK = 128


def _dist_argmin_body(z_ref, cbt_ref, zn_ref, cn_ref, idx_ref, dmin_ref,
                      rm_ref, rc_ref):
    t = pl.program_id(0)
    num_tiles = pl.num_programs(0) - 1
    hw = z_ref.shape[2]
    n = cbt_ref.shape[1]
    tiles_per_batch = hw // _BM

    @pl.when(t > 0)
    def _():
        rm = rm_ref[...]
        rc = rc_ref[...]
        fmin = jnp.min(rm, axis=1, keepdims=True)
        lane = jax.lax.broadcasted_iota(jnp.int32, (_BM, _CHUNK), 1)
        gidx = rc * _CHUNK + lane
        big = jnp.int32(2**31 - 1)
        fidx = jnp.min(jnp.where(rm == fmin, gidx, big), axis=1,
                       keepdims=True)
        idx_ref[0, 0, :] = jnp.transpose(fidx).reshape(_BM)
        dmin_ref[0, 0, :] = jnp.transpose(fmin).reshape(_BM)

    @pl.when(t < num_tiles)
    def _():
        b = t // tiles_per_batch
        chunk = t % tiles_per_batch
        zs = z_ref[b, :, pl.ds(chunk * _BM, _BM)]
        lhs = jnp.transpose(zs.astype(jnp.bfloat16))
        dot = jax.lax.dot_general(
            lhs, cbt_ref[...],
            dimension_numbers=(((1,), (0,)), ((), ())),
            preferred_element_type=jnp.float32,
        )
        zn = jnp.transpose(zn_ref[pl.ds(t, 1), :])

        def s_chunk(k):
            dk = jax.lax.slice(dot, (0, k * _CHUNK), (_BM, (k + 1) * _CHUNK))
            cnk = jax.lax.slice(cn_ref[...], (0, k * _CHUNK),
                                (1, (k + 1) * _CHUNK))
            return (zn + cnk) + dk

        rm = s_chunk(0)
        rc = jnp.zeros((_BM, _CHUNK), jnp.int32)
        for k in range(1, n // _CHUNK):
            v = s_chunk(k)
            mask = v < rm
            rm = jnp.where(mask, v, rm)
            rc = jnp.where(mask, jnp.int32(k), rc)
        rm_ref[...] = rm
        rc_ref[...] = rc


def _dist_argmin(z3, cbt, zn64, c_norm, interpret=False):
    b, c, hw = z3.shape
    m = b * hw
    n = cbt.shape[1]
    num_tiles = m // _BM
    grid = (num_tiles + 1,)
    prev = lambda t: (jnp.maximum(t - 1, 0), 0, 0)
    return pl.pallas_call(
        _dist_argmin_body,
        grid=grid,
        in_specs=[
            pl.BlockSpec(z3.shape, lambda t: (0, 0, 0)),
            pl.BlockSpec(cbt.shape, lambda t: (0, 0)),
            pl.BlockSpec(zn64.shape, lambda t: (0, 0)),
            pl.BlockSpec(c_norm.shape, lambda t: (0, 0)),
        ],
        out_specs=[
            pl.BlockSpec((1, 1, _BM), prev),
            pl.BlockSpec((1, 1, _BM), prev),
        ],
        out_shape=[
            jax.ShapeDtypeStruct((num_tiles, 1, _BM), jnp.int32),
            jax.ShapeDtypeStruct((num_tiles, 1, _BM), jnp.float32),
        ],
        scratch_shapes=[
            pltpu.VMEM((_BM, _CHUNK), jnp.float32),
            pltpu.VMEM((_BM, _CHUNK), jnp.int32),
        ],
        interpret=interpret,
    )(z3, cbt, zn64, c_norm)


_EP_W = 1024


def _st_transpose_body(z_ref, zq_ref, out_ref):
    zb = z_ref[0]
    zqt = jnp.transpose(zq_ref[...])
    out_ref[0] = zb + (zqt - zb)


def _st_transpose(z3, z_q_flat, interpret=False):
    b, c, hw = z3.shape
    j_steps = hw // _EP_W
    return pl.pallas_call(
        _st_transpose_body,
        grid=(b, j_steps),
        in_specs=[
            pl.BlockSpec((1, c, _EP_W), lambda t, j: (t, 0, j)),
            pl.BlockSpec((_EP_W, c), lambda t, j: (t * j_steps + j, 0)),
        ],
        out_specs=pl.BlockSpec((1, c, _EP_W), lambda t, j: (t, 0, j)),
        out_shape=jax.ShapeDtypeStruct((b, c, hw), jnp.float32),
        interpret=interpret,
    )(z3, z_q_flat)


_GATHER_W = 128


def _sc_gather(codebook, idx_flat):
    num_idx = idx_flat.shape[0]
    e_dim = codebook.shape[1]
    idx2 = idx_flat.reshape(1, num_idx)
    mesh = plsc.VectorSubcoreMesh(core_axis_name="core",
                                  subcore_axis_name="subcore")

    @functools.partial(
        pl.kernel,
        out_type=jax.ShapeDtypeStruct((num_idx, e_dim), codebook.dtype),
        mesh=mesh,
    )
    def gather_kernel(cb_hbm, i_hbm, o_hbm):
        def body(i_vmem, o_vmem):
            pltpu.sync_copy(cb_hbm.at[i_vmem.at[0]], o_vmem)

        pltpu.emit_pipeline(
            body,
            grid=(num_idx // _GATHER_W,),
            in_specs=[pl.BlockSpec((1, _GATHER_W), index_map=lambda i: (0, i))],
            out_specs=[pl.BlockSpec((_GATHER_W, e_dim),
                                    index_map=lambda i: (i, 0))],
            core_axis_name=("core", "subcore"),
            dimension_semantics=(pltpu.PARALLEL,),
        )(i_hbm, o_hbm)

    return gather_kernel(codebook, idx2)


def kernel(z, codebook):
    b, c, h, w = z.shape
    n_e = codebook.shape[0]
    z3 = z.reshape(b, c, h * w)
    zt = jnp.transpose(z, (0, 2, 3, 1))
    z_flat = zt.reshape(-1, c)
    z_norm = jnp.sum(z_flat ** 2, axis=1, keepdims=True)
    c_norm = jnp.sum(codebook ** 2, axis=1)[None, :]
    zn64 = z_norm.reshape(-1, _BM)

    cbt_bf = jnp.transpose((-2.0 * codebook).astype(jnp.bfloat16))

    idx3, dmin3 = _dist_argmin(z3, cbt_bf, zn64, c_norm)
    idx_flat = idx3.reshape(-1)

    z_q_flat = _sc_gather(codebook, idx_flat)
    z_q = _st_transpose(z3, z_q_flat).reshape(z.shape)

    mean_sq = jnp.sum(dmin3) / (b * h * w * c)
    loss = mean_sq + BETA * mean_sq
    idx = idx_flat.reshape(b, h, w)
    return z_q, loss, idx


# speedup vs baseline: 1.2652x; 1.0608x over previous
import functools

import jax
import jax.numpy as jnp
from jax.experimental import pallas as pl
from jax.experimental.pallas import tpu as pltpu
from jax.experimental.pallas import tpu_sc as plsc

BETA = 0.25
_BM = 128
_CHUNK = 128


def _dist_argmin_body(z_ref, cbt_ref, zn_ref, cn_ref, idx_ref, dmin_ref,
                      rm_ref, rc_ref):
    t = pl.program_id(0)
    num_tiles = pl.num_programs(0) - 1
    n = cbt_ref.shape[1]

    @pl.when(t > 0)
    def _():
        rm = rm_ref[...]
        rc = rc_ref[...]
        fmin = jnp.min(rm, axis=1, keepdims=True)
        lane = jax.lax.broadcasted_iota(jnp.int32, (_BM, _CHUNK), 1)
        gidx = rc * _CHUNK + lane
        big = jnp.int32(2**31 - 1)
        fidx = jnp.min(jnp.where(rm == fmin, gidx, big), axis=1,
                       keepdims=True)
        idx_ref[0, 0, :] = jnp.transpose(fidx).reshape(_BM)
        dmin_ref[0, 0, :] = jnp.transpose(fmin).reshape(_BM)

    @pl.when(t < num_tiles)
    def _():
        lhs = z_ref[pl.ds(t * _BM, _BM), :]
        dot = jax.lax.dot_general(
            lhs, cbt_ref[...],
            dimension_numbers=(((1,), (0,)), ((), ())),
            preferred_element_type=jnp.float32,
        )
        zn = jnp.transpose(zn_ref[pl.ds(t, 1), :])

        def s_chunk(k):
            dk = jax.lax.slice(dot, (0, k * _CHUNK), (_BM, (k + 1) * _CHUNK))
            cnk = jax.lax.slice(cn_ref[...], (0, k * _CHUNK),
                                (1, (k + 1) * _CHUNK))
            return (zn + cnk) + dk

        rm = s_chunk(0)
        rc = jnp.zeros((_BM, _CHUNK), jnp.int32)
        for k in range(1, n // _CHUNK):
            v = s_chunk(k)
            mask = v < rm
            rm = jnp.where(mask, v, rm)
            rc = jnp.where(mask, jnp.int32(k), rc)
        rm_ref[...] = rm
        rc_ref[...] = rc


def _dist_argmin(zf_bf, cbt, zn64, c_norm, interpret=False):
    m = zf_bf.shape[0]
    n = cbt.shape[1]
    num_tiles = m // _BM
    grid = (num_tiles + 1,)
    prev = lambda t: (jnp.maximum(t - 1, 0), 0, 0)
    return pl.pallas_call(
        _dist_argmin_body,
        grid=grid,
        in_specs=[
            pl.BlockSpec(zf_bf.shape, lambda t: (0, 0)),
            pl.BlockSpec(cbt.shape, lambda t: (0, 0)),
            pl.BlockSpec(zn64.shape, lambda t: (0, 0)),
            pl.BlockSpec(c_norm.shape, lambda t: (0, 0)),
        ],
        out_specs=[
            pl.BlockSpec((1, 1, _BM), prev),
            pl.BlockSpec((1, 1, _BM), prev),
        ],
        out_shape=[
            jax.ShapeDtypeStruct((num_tiles, 1, _BM), jnp.int32),
            jax.ShapeDtypeStruct((num_tiles, 1, _BM), jnp.float32),
        ],
        scratch_shapes=[
            pltpu.VMEM((_BM, _CHUNK), jnp.float32),
            pltpu.VMEM((_BM, _CHUNK), jnp.int32),
        ],
        interpret=interpret,
    )(zf_bf, cbt, zn64, c_norm)


_EP_W = 1024


def _st_transpose_body(z_ref, zq_ref, out_ref):
    zf = z_ref[...]
    zq = zq_ref[...]
    out_ref[0] = jnp.transpose(zf + (zq - zf))


def _st_transpose(z_flat, z_q_flat, b, hw, interpret=False):
    c = z_flat.shape[1]
    j_steps = (b * hw) // _EP_W
    per_b = hw // _EP_W
    return pl.pallas_call(
        _st_transpose_body,
        grid=(j_steps,),
        in_specs=[
            pl.BlockSpec((_EP_W, c), lambda t: (t, 0)),
            pl.BlockSpec((_EP_W, c), lambda t: (t, 0)),
        ],
        out_specs=pl.BlockSpec((1, c, _EP_W), lambda t: (t // per_b, 0,
                                                         t % per_b)),
        out_shape=jax.ShapeDtypeStruct((b, c, hw), jnp.float32),
        interpret=interpret,
    )(z_flat, z_q_flat)


_GATHER_W = 128


def _sc_gather(codebook, idx_flat):
    num_idx = idx_flat.shape[0]
    e_dim = codebook.shape[1]
    idx2 = idx_flat.reshape(1, num_idx)
    mesh = plsc.VectorSubcoreMesh(core_axis_name="core",
                                  subcore_axis_name="subcore")

    @functools.partial(
        pl.kernel,
        out_type=jax.ShapeDtypeStruct((num_idx, e_dim), codebook.dtype),
        mesh=mesh,
    )
    def gather_kernel(cb_hbm, i_hbm, o_hbm):
        def body(i_vmem, o_vmem):
            pltpu.sync_copy(cb_hbm.at[i_vmem.at[0]], o_vmem)

        pltpu.emit_pipeline(
            body,
            grid=(num_idx // _GATHER_W,),
            in_specs=[pl.BlockSpec((1, _GATHER_W), index_map=lambda i: (0, i))],
            out_specs=[pl.BlockSpec((_GATHER_W, e_dim),
                                    index_map=lambda i: (i, 0))],
            core_axis_name=("core", "subcore"),
            dimension_semantics=(pltpu.PARALLEL,),
        )(i_hbm, o_hbm)

    return gather_kernel(codebook, idx2)


def kernel(z, codebook):
    b, c, h, w = z.shape
    n_e = codebook.shape[0]
    zt = jnp.transpose(z, (0, 2, 3, 1))
    z_flat = zt.reshape(-1, c)
    z_norm = jnp.sum(z_flat ** 2, axis=1, keepdims=True)
    c_norm = jnp.sum(codebook ** 2, axis=1)[None, :]
    zn64 = z_norm.reshape(-1, _BM)

    zf_bf = z_flat.astype(jnp.bfloat16)
    cbt_bf = jnp.transpose((-2.0 * codebook).astype(jnp.bfloat16))

    idx3, dmin3 = _dist_argmin(zf_bf, cbt_bf, zn64, c_norm)
    idx_flat = idx3.reshape(-1)

    z_q_flat = _sc_gather(codebook, idx_flat)
    z_q = _st_transpose(z_flat, z_q_flat, b, h * w).reshape(z.shape)

    mean_sq = jnp.sum(dmin3) / (b * h * w * c)
    loss = mean_sq + BETA * mean_sq
    idx = idx_flat.reshape(b, h, w)
    return z_q, loss, idx


# speedup vs baseline: 1.2884x; 1.0184x over previous
import functools

import jax
import jax.numpy as jnp
from jax.experimental import pallas as pl
from jax.experimental.pallas import tpu as pltpu
from jax.experimental.pallas import tpu_sc as plsc

BETA = 0.25
_BM = 128
_CHUNK = 128


def _dist_argmin_body(z_ref, cbt_ref, zn_ref, cn_ref, idx_ref, dmin_ref,
                      rm_ref, rc_ref):
    t = pl.program_id(0)
    num_tiles = pl.num_programs(0) - 1
    n = cbt_ref.shape[1]

    @pl.when(t > 0)
    def _():
        rm = rm_ref[...]
        rc = rc_ref[...]
        fmin = jnp.min(rm, axis=1, keepdims=True)
        lane = jax.lax.broadcasted_iota(jnp.int32, (_BM, _CHUNK), 1)
        gidx = rc * _CHUNK + lane
        big = jnp.int32(2**31 - 1)
        fidx = jnp.min(jnp.where(rm == fmin, gidx, big), axis=1,
                       keepdims=True)
        idx_ref[0, 0, :] = jnp.transpose(fidx).reshape(_BM)
        dmin_ref[0, 0, :] = jnp.transpose(fmin).reshape(_BM)

    @pl.when(t < num_tiles)
    def _():
        lhs = z_ref[pl.ds(t * _BM, _BM), :]
        dot = jax.lax.dot_general(
            lhs, cbt_ref[...],
            dimension_numbers=(((1,), (0,)), ((), ())),
            preferred_element_type=jnp.float32,
        )
        zn = jnp.transpose(zn_ref[pl.ds(t, 1), :])

        def s_chunk(k):
            dk = jax.lax.slice(dot, (0, k * _CHUNK), (_BM, (k + 1) * _CHUNK))
            cnk = jax.lax.slice(cn_ref[...], (0, k * _CHUNK),
                                (1, (k + 1) * _CHUNK))
            return (zn + cnk) + dk

        rm = s_chunk(0)
        rc = jnp.zeros((_BM, _CHUNK), jnp.int32)
        for k in range(1, n // _CHUNK):
            v = s_chunk(k)
            mask = v < rm
            rm = jnp.where(mask, v, rm)
            rc = jnp.where(mask, jnp.int32(k), rc)
        rm_ref[...] = rm
        rc_ref[...] = rc


def _dist_argmin(zf_bf, cbt, zn64, c_norm, interpret=False):
    m = zf_bf.shape[0]
    n = cbt.shape[1]
    num_tiles = m // _BM
    grid = (num_tiles + 1,)
    prev = lambda t: (jnp.maximum(t - 1, 0), 0, 0)
    return pl.pallas_call(
        _dist_argmin_body,
        grid=grid,
        in_specs=[
            pl.BlockSpec(zf_bf.shape, lambda t: (0, 0)),
            pl.BlockSpec(cbt.shape, lambda t: (0, 0)),
            pl.BlockSpec(zn64.shape, lambda t: (0, 0)),
            pl.BlockSpec(c_norm.shape, lambda t: (0, 0)),
        ],
        out_specs=[
            pl.BlockSpec((1, 1, _BM), prev),
            pl.BlockSpec((1, 1, _BM), prev),
        ],
        out_shape=[
            jax.ShapeDtypeStruct((num_tiles, 1, _BM), jnp.int32),
            jax.ShapeDtypeStruct((num_tiles, 1, _BM), jnp.float32),
        ],
        scratch_shapes=[
            pltpu.VMEM((_BM, _CHUNK), jnp.float32),
            pltpu.VMEM((_BM, _CHUNK), jnp.int32),
        ],
        interpret=interpret,
    )(zf_bf, cbt, zn64, c_norm)


_EP_W = 1024


def _st_transpose_body(z_ref, zq_ref, out_ref):
    zf = z_ref[...]
    zq = zq_ref[...]
    out_ref[0] = jnp.transpose(zf + (zq - zf))


def _st_transpose(z_flat, z_q_flat, b, hw, interpret=False):
    c = z_flat.shape[1]
    j_steps = (b * hw) // _EP_W
    per_b = hw // _EP_W
    return pl.pallas_call(
        _st_transpose_body,
        grid=(j_steps,),
        in_specs=[
            pl.BlockSpec((_EP_W, c), lambda t: (t, 0)),
            pl.BlockSpec((_EP_W, c), lambda t: (t, 0)),
        ],
        out_specs=pl.BlockSpec((1, c, _EP_W), lambda t: (t // per_b, 0,
                                                         t % per_b)),
        out_shape=jax.ShapeDtypeStruct((b, c, hw), jnp.float32),
        interpret=interpret,
    )(z_flat, z_q_flat)


_GATHER_W = 128


def _sc_gather(codebook, idx_flat):
    num_idx = idx_flat.shape[0]
    e_dim = codebook.shape[1]
    idx2 = idx_flat.reshape(1, num_idx)
    mesh = plsc.VectorSubcoreMesh(core_axis_name="core",
                                  subcore_axis_name="subcore")

    @functools.partial(
        pl.kernel,
        out_type=jax.ShapeDtypeStruct((num_idx, e_dim), codebook.dtype),
        mesh=mesh,
    )
    def gather_kernel(cb_hbm, i_hbm, o_hbm):
        def body(i_vmem, o_vmem):
            pltpu.sync_copy(cb_hbm.at[i_vmem.at[0]], o_vmem)

        pltpu.emit_pipeline(
            body,
            grid=(num_idx // _GATHER_W,),
            in_specs=[pl.BlockSpec((1, _GATHER_W), index_map=lambda i: (0, i))],
            out_specs=[pl.BlockSpec((_GATHER_W, e_dim),
                                    index_map=lambda i: (i, 0))],
            core_axis_name=("core", "subcore"),
            dimension_semantics=(pltpu.PARALLEL,),
        )(i_hbm, o_hbm)

    return gather_kernel(codebook, idx2)


def kernel(z, codebook):
    b, c, h, w = z.shape
    n_e = codebook.shape[0]
    zt = jnp.transpose(z, (0, 2, 3, 1))
    z_flat = jax.lax.optimization_barrier(zt.reshape(-1, c))
    z_norm = jnp.sum(z_flat ** 2, axis=1, keepdims=True)
    c_norm = jnp.sum(codebook ** 2, axis=1)[None, :]
    zn64 = z_norm.reshape(-1, _BM)

    zf_bf = z_flat.astype(jnp.bfloat16)
    cbt_bf = jnp.transpose((-2.0 * codebook).astype(jnp.bfloat16))

    idx3, dmin3 = _dist_argmin(zf_bf, cbt_bf, zn64, c_norm)
    idx_flat = idx3.reshape(-1)

    z_q_flat = _sc_gather(codebook, idx_flat)
    z_q = _st_transpose(z_flat, z_q_flat, b, h * w).reshape(z.shape)

    mean_sq = jnp.sum(dmin3) / (b * h * w * c)
    loss = mean_sq + BETA * mean_sq
    idx = idx_flat.reshape(b, h, w)
    return z_q, loss, idx


# speedup vs baseline: 1.3559x; 1.0524x over previous
import functools

import jax
import jax.numpy as jnp
from jax.experimental import pallas as pl
from jax.experimental.pallas import tpu as pltpu
from jax.experimental.pallas import tpu_sc as plsc

BETA = 0.25
_BM = 128
_CHUNK = 128


def _dist_argmin_body(z_ref, cb_ref, zn_ref, cn_ref, idx_ref, dmin_ref,
                      cbt_ref, rm_ref, rc_ref):
    t = pl.program_id(0)
    num_tiles = pl.num_programs(0) - 1
    n = cbt_ref.shape[1]

    @pl.when(t == 0)
    def _():
        cbt_ref[...] = jnp.transpose((-2.0 * cb_ref[...]).astype(jnp.bfloat16))

    @pl.when(t > 0)
    def _():
        rm = rm_ref[...]
        rc = rc_ref[...]
        fmin = jnp.min(rm, axis=1, keepdims=True)
        lane = jax.lax.broadcasted_iota(jnp.int32, (_BM, _CHUNK), 1)
        gidx = rc * _CHUNK + lane
        big = jnp.int32(2**31 - 1)
        fidx = jnp.min(jnp.where(rm == fmin, gidx, big), axis=1,
                       keepdims=True)
        idx_ref[0, 0, :] = jnp.transpose(fidx).reshape(_BM)
        dmin_ref[0, 0, :] = jnp.transpose(fmin).reshape(_BM)

    @pl.when(t < num_tiles)
    def _():
        lhs = z_ref[pl.ds(t * _BM, _BM), :]
        dot = jax.lax.dot_general(
            lhs, cbt_ref[...],
            dimension_numbers=(((1,), (0,)), ((), ())),
            preferred_element_type=jnp.float32,
        )
        zn = jnp.transpose(zn_ref[pl.ds(t, 1), :])

        def s_chunk(k):
            dk = jax.lax.slice(dot, (0, k * _CHUNK), (_BM, (k + 1) * _CHUNK))
            cnk = jax.lax.slice(cn_ref[...], (0, k * _CHUNK),
                                (1, (k + 1) * _CHUNK))
            return (zn + cnk) + dk

        rm = s_chunk(0)
        rc = jnp.zeros((_BM, _CHUNK), jnp.int32)
        for k in range(1, n // _CHUNK):
            v = s_chunk(k)
            mask = v < rm
            rm = jnp.where(mask, v, rm)
            rc = jnp.where(mask, jnp.int32(k), rc)
        rm_ref[...] = rm
        rc_ref[...] = rc


def _dist_argmin(zf_bf, codebook, zn64, c_norm, interpret=False):
    m = zf_bf.shape[0]
    n = codebook.shape[0]
    num_tiles = m // _BM
    grid = (num_tiles + 1,)
    prev = lambda t: (jnp.maximum(t - 1, 0), 0, 0)
    return pl.pallas_call(
        _dist_argmin_body,
        grid=grid,
        in_specs=[
            pl.BlockSpec(zf_bf.shape, lambda t: (0, 0)),
            pl.BlockSpec(codebook.shape, lambda t: (0, 0)),
            pl.BlockSpec(zn64.shape, lambda t: (0, 0)),
            pl.BlockSpec(c_norm.shape, lambda t: (0, 0)),
        ],
        out_specs=[
            pl.BlockSpec((1, 1, _BM), prev),
            pl.BlockSpec((1, 1, _BM), prev),
        ],
        out_shape=[
            jax.ShapeDtypeStruct((num_tiles, 1, _BM), jnp.int32),
            jax.ShapeDtypeStruct((num_tiles, 1, _BM), jnp.float32),
        ],
        scratch_shapes=[
            pltpu.VMEM((codebook.shape[1], n), jnp.bfloat16),
            pltpu.VMEM((_BM, _CHUNK), jnp.float32),
            pltpu.VMEM((_BM, _CHUNK), jnp.int32),
        ],
        interpret=interpret,
    )(zf_bf, codebook, zn64, c_norm)


_EP_W = 1024


def _st_transpose_body(z_ref, zq_ref, out_ref):
    zf = z_ref[...]
    zq = zq_ref[...]
    out_ref[0] = jnp.transpose(zf + (zq - zf))


def _st_transpose(z_flat, z_q_flat, b, hw, interpret=False):
    c = z_flat.shape[1]
    j_steps = (b * hw) // _EP_W
    per_b = hw // _EP_W
    return pl.pallas_call(
        _st_transpose_body,
        grid=(j_steps,),
        in_specs=[
            pl.BlockSpec((_EP_W, c), lambda t: (t, 0)),
            pl.BlockSpec((_EP_W, c), lambda t: (t, 0)),
        ],
        out_specs=pl.BlockSpec((1, c, _EP_W), lambda t: (t // per_b, 0,
                                                         t % per_b)),
        out_shape=jax.ShapeDtypeStruct((b, c, hw), jnp.float32),
        interpret=interpret,
    )(z_flat, z_q_flat)


_GATHER_W = 128


def _sc_gather(codebook, idx_flat):
    num_idx = idx_flat.shape[0]
    e_dim = codebook.shape[1]
    idx2 = idx_flat.reshape(1, num_idx)
    mesh = plsc.VectorSubcoreMesh(core_axis_name="core",
                                  subcore_axis_name="subcore")

    @functools.partial(
        pl.kernel,
        out_type=jax.ShapeDtypeStruct((num_idx, e_dim), codebook.dtype),
        mesh=mesh,
    )
    def gather_kernel(cb_hbm, i_hbm, o_hbm):
        def body(i_vmem, o_vmem):
            pltpu.sync_copy(cb_hbm.at[i_vmem.at[0]], o_vmem)

        pltpu.emit_pipeline(
            body,
            grid=(num_idx // _GATHER_W,),
            in_specs=[pl.BlockSpec((1, _GATHER_W), index_map=lambda i: (0, i))],
            out_specs=[pl.BlockSpec((_GATHER_W, e_dim),
                                    index_map=lambda i: (i, 0))],
            core_axis_name=("core", "subcore"),
            dimension_semantics=(pltpu.PARALLEL,),
        )(i_hbm, o_hbm)

    return gather_kernel(codebook, idx2)


def kernel(z, codebook):
    b, c, h, w = z.shape
    n_e = codebook.shape[0]
    zt = jnp.transpose(z, (0, 2, 3, 1))
    z_flat = jax.lax.optimization_barrier(zt.reshape(-1, c))
    z_norm = jnp.sum(z_flat ** 2, axis=1, keepdims=True)
    c_norm = jnp.sum(codebook ** 2, axis=1)[None, :]
    zn64 = z_norm.reshape(-1, _BM)

    zf_bf = z_flat.astype(jnp.bfloat16)

    idx3, dmin3 = _dist_argmin(zf_bf, codebook, zn64, c_norm)
    idx_flat = idx3.reshape(-1)

    z_q_flat = _sc_gather(codebook, idx_flat)
    z_q = _st_transpose(z_flat, z_q_flat, b, h * w).reshape(z.shape)

    mean_sq = jnp.sum(dmin3) / (b * h * w * c)
    loss = mean_sq + BETA * mean_sq
    idx = idx_flat.reshape(b, h, w)
    return z_q, loss, idx
